# Initial kernel scaffold; baseline (speedup 1.0000x reference)
import jax
import jax.numpy as jnp
from jax.experimental import pallas as pl


def kernel(wordid, x, h, c, emb, W_iou_w, W_iou_b, U_iou_w, U_iou_b, U_f_w, U_f_b, lin_w, lin_b):
    raise NotImplementedError("write your pallas kernel here")


# speedup vs baseline: 3.9743x; 3.9743x over previous
import functools

import jax
import jax.numpy as jnp
from jax import lax
from jax.experimental import pallas as pl
from jax.experimental.pallas import tpu as pltpu
from jax.experimental.pallas import tpu_sc as plsc

H = 256


def _sc_gather(table, idx):
    B = idx.shape[0]
    D = table.shape[1]
    NW = 32
    b_per_w = B // NW
    CH = 128
    n_ch = b_per_w // CH

    mesh = plsc.VectorSubcoreMesh(core_axis_name="c", subcore_axis_name="s")

    @functools.partial(
        pl.kernel, mesh=mesh,
        out_type=jax.ShapeDtypeStruct((B, D), jnp.float32),
        scratch_types=[
            pltpu.VMEM((CH,), jnp.int32),
            pltpu.VMEM((CH, D), jnp.float32),
            pltpu.SemaphoreType.DMA,
        ],
    )
    def k(table_hbm, idx_hbm, out_hbm, idx_v, rows_v, sem):
        wid = lax.axis_index("s") * 2 + lax.axis_index("c")
        for ch in range(n_ch):
            base = wid * b_per_w + ch * CH
            pltpu.sync_copy(idx_hbm.at[pl.ds(base, CH)], idx_v)
            pltpu.async_copy(table_hbm.at[idx_v], rows_v, sem).wait()
            pltpu.sync_copy(rows_v, out_hbm.at[pl.ds(base, CH)])

    return k(table, idx)


def _leaf_body(e_ref, w_ref, b_ref, lw_ref, lb_ref, h_ref, c_ref, lg_ref):
    z = jnp.dot(e_ref[...], w_ref[...],
                preferred_element_type=jnp.float32) + b_ref[...]
    i = jax.nn.sigmoid(z[:, :H])
    o = jax.nn.sigmoid(z[:, H:2 * H])
    u = jnp.tanh(z[:, 2 * H:])
    c0 = i * u
    h0 = o * jnp.tanh(c0)
    h_ref[...] = h0
    c_ref[...] = c0
    lg_ref[...] = jnp.dot(h0, lw_ref[...],
                          preferred_element_type=jnp.float32) + lb_ref[...]


def _leaf(embeds, W_iou_w, W_iou_b, lin_w, lin_b, blk):
    n = embeds.shape[0]
    X = embeds.shape[1]
    NC = lin_w.shape[1]
    grid = (n // blk,)
    return pl.pallas_call(
        _leaf_body,
        grid=grid,
        in_specs=[
            pl.BlockSpec((blk, X), lambda j: (j, 0)),
            pl.BlockSpec((X, 3 * H), lambda j: (0, 0)),
            pl.BlockSpec((1, 3 * H), lambda j: (0, 0)),
            pl.BlockSpec((H, NC), lambda j: (0, 0)),
            pl.BlockSpec((1, NC), lambda j: (0, 0)),
        ],
        out_specs=[
            pl.BlockSpec((blk, H), lambda j: (j, 0)),
            pl.BlockSpec((blk, H), lambda j: (j, 0)),
            pl.BlockSpec((blk, NC), lambda j: (j, 0)),
        ],
        out_shape=[
            jax.ShapeDtypeStruct((n, H), jnp.float32),
            jax.ShapeDtypeStruct((n, H), jnp.float32),
            jax.ShapeDtypeStruct((n, NC), jnp.float32),
        ],
    )(embeds, W_iou_w, W_iou_b, lin_w, lin_b)


def _level_body(h2_ref, c2_ref, u_ref, b_ref, lw_ref, lb_ref,
                h_ref, c_ref, lg_ref):
    z = jnp.dot(h2_ref[...], u_ref[...],
                preferred_element_type=jnp.float32) + b_ref[...]
    f = jax.nn.sigmoid(z[:, :2 * H])
    c2 = c2_ref[...]
    cf = f[:, :H] * c2[:, :H] + f[:, H:] * c2[:, H:]
    i = jax.nn.sigmoid(z[:, 2 * H:3 * H])
    o = jax.nn.sigmoid(z[:, 3 * H:4 * H])
    u = jnp.tanh(z[:, 4 * H:])
    c_new = i * u + cf
    h_new = o * jnp.tanh(c_new)
    h_ref[...] = h_new
    c_ref[...] = c_new
    lg_ref[...] = jnp.dot(h_new, lw_ref[...],
                          preferred_element_type=jnp.float32) + lb_ref[...]


def _level(h2, c2, U_cat, b_cat, lin_w, lin_b, blk):
    n = h2.shape[0]
    NC = lin_w.shape[1]
    grid = (n // blk,)
    return pl.pallas_call(
        _level_body,
        grid=grid,
        in_specs=[
            pl.BlockSpec((blk, 2 * H), lambda j: (j, 0)),
            pl.BlockSpec((blk, 2 * H), lambda j: (j, 0)),
            pl.BlockSpec((2 * H, 5 * H), lambda j: (0, 0)),
            pl.BlockSpec((1, 5 * H), lambda j: (0, 0)),
            pl.BlockSpec((H, NC), lambda j: (0, 0)),
            pl.BlockSpec((1, NC), lambda j: (0, 0)),
        ],
        out_specs=[
            pl.BlockSpec((blk, H), lambda j: (j, 0)),
            pl.BlockSpec((blk, H), lambda j: (j, 0)),
            pl.BlockSpec((blk, NC), lambda j: (j, 0)),
        ],
        out_shape=[
            jax.ShapeDtypeStruct((n, H), jnp.float32),
            jax.ShapeDtypeStruct((n, H), jnp.float32),
            jax.ShapeDtypeStruct((n, NC), jnp.float32),
        ],
    )(h2, c2, U_cat, b_cat, lin_w, lin_b)


def kernel(wordid, x, h, c, emb, W_iou_w, W_iou_b, U_iou_w, U_iou_b,
           U_f_w, U_f_b, lin_w, lin_b):
    del x, h, c

    embeds = _sc_gather(emb, wordid.astype(jnp.int32))

    h_prev, c_prev, lg0 = _leaf(
        embeds, W_iou_w, W_iou_b.reshape(1, -1), lin_w,
        lin_b.reshape(1, -1), blk=1024)
    logits = [lg0]

    U_cat = jnp.concatenate([U_f_w, U_iou_w], axis=1)
    b_cat = jnp.concatenate([U_f_b, U_iou_b]).reshape(1, -1)

    for _ in range(7):
        n_l = h_prev.shape[0] // 2
        h2 = h_prev.reshape(n_l, 2 * H)
        c2 = c_prev.reshape(n_l, 2 * H)
        h_prev, c_prev, lg = _level(
            h2, c2, U_cat, b_cat, lin_w, lin_b.reshape(1, -1),
            blk=min(n_l, 512))
        logits.append(lg)

    return jnp.concatenate(logits, axis=0)


# speedup vs baseline: 4.3420x; 1.0925x over previous
import functools

import jax
import jax.numpy as jnp
from jax import lax
from jax.experimental import pallas as pl
from jax.experimental.pallas import tpu as pltpu
from jax.experimental.pallas import tpu_sc as plsc

H = 256


def _sc_gather(table, idx):
    B = idx.shape[0]
    D = table.shape[1]
    NW = 32
    b_per_w = B // NW
    CH = 128
    n_ch = b_per_w // CH

    mesh = plsc.VectorSubcoreMesh(core_axis_name="c", subcore_axis_name="s")

    @functools.partial(
        pl.kernel, mesh=mesh,
        out_type=jax.ShapeDtypeStruct((B, D), jnp.float32),
        scratch_types=[
            pltpu.VMEM((CH,), jnp.int32),
            pltpu.VMEM((CH, D), jnp.float32),
            pltpu.SemaphoreType.DMA,
        ],
    )
    def k(table_hbm, idx_hbm, out_hbm, idx_v, rows_v, sem):
        wid = lax.axis_index("s") * 2 + lax.axis_index("c")
        for ch in range(n_ch):
            base = wid * b_per_w + ch * CH
            pltpu.sync_copy(idx_hbm.at[pl.ds(base, CH)], idx_v)
            pltpu.async_copy(table_hbm.at[idx_v], rows_v, sem).wait()
            pltpu.sync_copy(rows_v, out_hbm.at[pl.ds(base, CH)])

    return k(table, idx)


def _leaf_body(e_ref, w_ref, b_ref, lw_ref, lb_ref, h_ref, c_ref, lg_ref):
    z = jnp.dot(e_ref[...].astype(jnp.bfloat16), w_ref[...],
                preferred_element_type=jnp.float32) + b_ref[...]
    i = jax.nn.sigmoid(z[:, :H])
    o = jax.nn.sigmoid(z[:, H:2 * H])
    u = jnp.tanh(z[:, 2 * H:])
    c0 = i * u
    h0 = (o * jnp.tanh(c0)).astype(jnp.bfloat16)
    h_ref[...] = h0
    c_ref[...] = c0
    lg_ref[...] = jnp.dot(h0, lw_ref[...],
                          preferred_element_type=jnp.float32) + lb_ref[...]


def _leaf(embeds, W_iou_w, W_iou_b, lin_w, lin_b, blk):
    n = embeds.shape[0]
    X = embeds.shape[1]
    NC = lin_w.shape[1]
    grid = (n // blk,)
    return pl.pallas_call(
        _leaf_body,
        grid=grid,
        in_specs=[
            pl.BlockSpec((blk, X), lambda j: (j, 0)),
            pl.BlockSpec((X, 3 * H), lambda j: (0, 0)),
            pl.BlockSpec((1, 3 * H), lambda j: (0, 0)),
            pl.BlockSpec((H, NC), lambda j: (0, 0)),
            pl.BlockSpec((1, NC), lambda j: (0, 0)),
        ],
        out_specs=[
            pl.BlockSpec((blk, H), lambda j: (j, 0)),
            pl.BlockSpec((blk, H), lambda j: (j, 0)),
            pl.BlockSpec((blk, NC), lambda j: (j, 0)),
        ],
        out_shape=[
            jax.ShapeDtypeStruct((n, H), jnp.bfloat16),
            jax.ShapeDtypeStruct((n, H), jnp.float32),
            jax.ShapeDtypeStruct((n, NC), jnp.float32),
        ],
    )(embeds, W_iou_w, W_iou_b, lin_w, lin_b)


def _level_body(h2_ref, c2_ref, u_ref, b_ref, lw_ref, lb_ref,
                h_ref, c_ref, lg_ref):
    z = jnp.dot(h2_ref[...], u_ref[...],
                preferred_element_type=jnp.float32) + b_ref[...]
    f = jax.nn.sigmoid(z[:, :2 * H])
    c2 = c2_ref[...]
    cf = f[:, :H] * c2[:, :H] + f[:, H:] * c2[:, H:]
    i = jax.nn.sigmoid(z[:, 2 * H:3 * H])
    o = jax.nn.sigmoid(z[:, 3 * H:4 * H])
    u = jnp.tanh(z[:, 4 * H:])
    c_new = i * u + cf
    h_new = (o * jnp.tanh(c_new)).astype(jnp.bfloat16)
    h_ref[...] = h_new
    c_ref[...] = c_new
    lg_ref[...] = jnp.dot(h_new, lw_ref[...],
                          preferred_element_type=jnp.float32) + lb_ref[...]


def _level(h2, c2, U_cat, b_cat, lin_w, lin_b, blk):
    n = h2.shape[0]
    NC = lin_w.shape[1]
    grid = (n // blk,)
    return pl.pallas_call(
        _level_body,
        grid=grid,
        in_specs=[
            pl.BlockSpec((blk, 2 * H), lambda j: (j, 0)),
            pl.BlockSpec((blk, 2 * H), lambda j: (j, 0)),
            pl.BlockSpec((2 * H, 5 * H), lambda j: (0, 0)),
            pl.BlockSpec((1, 5 * H), lambda j: (0, 0)),
            pl.BlockSpec((H, NC), lambda j: (0, 0)),
            pl.BlockSpec((1, NC), lambda j: (0, 0)),
        ],
        out_specs=[
            pl.BlockSpec((blk, H), lambda j: (j, 0)),
            pl.BlockSpec((blk, H), lambda j: (j, 0)),
            pl.BlockSpec((blk, NC), lambda j: (j, 0)),
        ],
        out_shape=[
            jax.ShapeDtypeStruct((n, H), jnp.bfloat16),
            jax.ShapeDtypeStruct((n, H), jnp.float32),
            jax.ShapeDtypeStruct((n, NC), jnp.float32),
        ],
    )(h2, c2, U_cat, b_cat, lin_w, lin_b)


def kernel(wordid, x, h, c, emb, W_iou_w, W_iou_b, U_iou_w, U_iou_b,
           U_f_w, U_f_b, lin_w, lin_b):
    del x, h, c

    embeds = _sc_gather(emb, wordid.astype(jnp.int32))

    W_iou_w = W_iou_w.astype(jnp.bfloat16)
    lin_w = lin_w.astype(jnp.bfloat16)

    h_prev, c_prev, lg0 = _leaf(
        embeds, W_iou_w, W_iou_b.reshape(1, -1), lin_w,
        lin_b.reshape(1, -1), blk=1024)
    logits = [lg0]

    U_cat = jnp.concatenate([U_f_w, U_iou_w],
                            axis=1).astype(jnp.bfloat16)
    b_cat = jnp.concatenate([U_f_b, U_iou_b]).reshape(1, -1)

    for _ in range(7):
        n_l = h_prev.shape[0] // 2
        h2 = h_prev.reshape(n_l, 2 * H)
        c2 = c_prev.reshape(n_l, 2 * H)
        h_prev, c_prev, lg = _level(
            h2, c2, U_cat, b_cat, lin_w, lin_b.reshape(1, -1),
            blk=min(n_l, 512))
        logits.append(lg)

    return jnp.concatenate(logits, axis=0)


# speedup vs baseline: 7.8046x; 1.7974x over previous
import functools

import jax
import jax.numpy as jnp
from jax import lax
from jax.experimental import pallas as pl
from jax.experimental.pallas import tpu as pltpu
from jax.experimental.pallas import tpu_sc as plsc

H = 256
N_LEAVES = 16384
DEPTH = 8
LEVEL_SIZES = [N_LEAVES >> l for l in range(DEPTH)]
N_TOTAL = sum(LEVEL_SIZES)
LEAF_BLK = 512
LVL_BLK = 512


def _sc_gather(table, idx):
    B = idx.shape[0]
    D = table.shape[1]
    NW = 32
    b_per_w = B // NW
    CH = 128
    n_ch = b_per_w // CH

    mesh = plsc.VectorSubcoreMesh(core_axis_name="c", subcore_axis_name="s")

    @functools.partial(
        pl.kernel, mesh=mesh,
        out_type=jax.ShapeDtypeStruct((B, D), jnp.float32),
        scratch_types=[
            pltpu.VMEM((CH,), jnp.int32),
            pltpu.VMEM((CH, D), jnp.float32),
            pltpu.SemaphoreType.DMA,
        ],
    )
    def k(table_hbm, idx_hbm, out_hbm, idx_v, rows_v, sem):
        wid = lax.axis_index("s") * 2 + lax.axis_index("c")
        for ch in range(n_ch):
            base = wid * b_per_w + ch * CH
            pltpu.sync_copy(idx_hbm.at[pl.ds(base, CH)], idx_v)
            pltpu.async_copy(table_hbm.at[idx_v], rows_v, sem).wait()
            pltpu.sync_copy(rows_v, out_hbm.at[pl.ds(base, CH)])

    return k(table, idx)


def _tree_body(emb_hbm, wiou_ref, biou_ref, ucat_ref, bcat_ref, lw_ref,
               lb_ref, lg_ref, ebuf, hA, hB, cA, cB, sems):
    def emb_copy(i, slot):
        return pltpu.make_async_copy(
            emb_hbm.at[pl.ds(pl.multiple_of(i * LEAF_BLK, LEAF_BLK),
                             LEAF_BLK), :],
            ebuf.at[slot], sems.at[slot])

    def leaf_block(i, slot):
        emb_copy(i, slot).wait()
        e = ebuf[slot].astype(jnp.bfloat16)
        z = jnp.dot(e, wiou_ref[...],
                    preferred_element_type=jnp.float32) + biou_ref[...]
        ig = jax.nn.sigmoid(z[:, :H])
        og = jax.nn.sigmoid(z[:, H:2 * H])
        ug = jnp.tanh(z[:, 2 * H:])
        c0 = ig * ug
        h0 = (og * jnp.tanh(c0)).astype(jnp.bfloat16)
        half = pl.multiple_of(i * (LEAF_BLK // 2), LEAF_BLK // 2)
        hA[pl.ds(half, LEAF_BLK // 2), :] = h0.reshape(LEAF_BLK // 2, 2 * H)
        cA[pl.ds(half, LEAF_BLK // 2), :] = c0.reshape(LEAF_BLK // 2, 2 * H)
        lg_ref[pl.ds(pl.multiple_of(i * LEAF_BLK, LEAF_BLK),
                     LEAF_BLK), :] = jnp.dot(
            h0, lw_ref[...], preferred_element_type=jnp.float32) + lb_ref[...]

    n_leaf_blk = N_LEAVES // LEAF_BLK
    emb_copy(0, 0).start()

    def leaf_pair(p, carry):
        i0 = 2 * p
        emb_copy(i0 + 1, 1).start()
        leaf_block(i0, 0)

        @pl.when(i0 + 2 < n_leaf_blk)
        def _():
            emb_copy(i0 + 2, 0).start()
        leaf_block(i0 + 1, 1)
        return carry

    lax.fori_loop(0, n_leaf_blk // 2, leaf_pair, 0)

    def level_block(h_in, c_in, h_out, c_out, blk, jb, off, last):
        jb = pl.multiple_of(jb, blk)
        h2 = h_in[pl.ds(jb, blk), :]
        c2 = c_in[pl.ds(jb, blk), :]
        z = jnp.dot(h2, ucat_ref[...],
                    preferred_element_type=jnp.float32) + bcat_ref[...]
        f = jax.nn.sigmoid(z[:, :2 * H])
        cf = f[:, :H] * c2[:, :H] + f[:, H:] * c2[:, H:]
        ig = jax.nn.sigmoid(z[:, 2 * H:3 * H])
        og = jax.nn.sigmoid(z[:, 3 * H:4 * H])
        ug = jnp.tanh(z[:, 4 * H:])
        c_new = ig * ug + cf
        h_new = (og * jnp.tanh(c_new)).astype(jnp.bfloat16)
        if not last:
            jh = pl.multiple_of(jb // 2, blk // 2)
            h_out[pl.ds(jh, blk // 2), :] = h_new.reshape(blk // 2, 2 * H)
            c_out[pl.ds(jh, blk // 2), :] = c_new.reshape(blk // 2, 2 * H)
        lg_ref[pl.ds(pl.multiple_of(off + jb, blk), blk), :] = jnp.dot(
            h_new, lw_ref[...],
            preferred_element_type=jnp.float32) + lb_ref[...]

    off = N_LEAVES
    bufs = [(hA, cA), (hB, cB)]
    for l in range(1, DEPTH):
        n_l = LEVEL_SIZES[l]
        h_in, c_in = bufs[(l - 1) % 2]
        h_out, c_out = bufs[l % 2]
        blk = min(n_l, LVL_BLK)
        n_blk = n_l // blk
        last = l == DEPTH - 1
        off_l = off
        if n_blk > 1:
            def level_step(j, carry, h_in=h_in, c_in=c_in, h_out=h_out,
                           c_out=c_out, blk=blk, off_l=off_l, last=last):
                level_block(h_in, c_in, h_out, c_out, blk, j * blk,
                            off_l, last)
                return carry
            lax.fori_loop(0, n_blk, level_step, 0)
        else:
            level_block(h_in, c_in, h_out, c_out, blk, 0, off_l, last)
        off += n_l


def _tree(embeds, W_iou_w, W_iou_b, U_cat, b_cat, lin_w, lin_b):
    NC = lin_w.shape[1]
    vmem = pl.BlockSpec(memory_space=pltpu.MemorySpace.VMEM)
    return pl.pallas_call(
        _tree_body,
        in_specs=[
            pl.BlockSpec(memory_space=pltpu.MemorySpace.HBM),
            vmem, vmem, vmem, vmem, vmem, vmem,
        ],
        out_specs=pl.BlockSpec(memory_space=pltpu.MemorySpace.VMEM),
        out_shape=jax.ShapeDtypeStruct((N_TOTAL, NC), jnp.float32),
        scratch_shapes=[
            pltpu.VMEM((2, LEAF_BLK, H), jnp.float32),
            pltpu.VMEM((N_LEAVES // 2, 2 * H), jnp.bfloat16),
            pltpu.VMEM((N_LEAVES // 4, 2 * H), jnp.bfloat16),
            pltpu.VMEM((N_LEAVES // 2, 2 * H), jnp.float32),
            pltpu.VMEM((N_LEAVES // 4, 2 * H), jnp.float32),
            pltpu.SemaphoreType.DMA((2,)),
        ],
    )(embeds, W_iou_w, W_iou_b, U_cat, b_cat, lin_w, lin_b)


def kernel(wordid, x, h, c, emb, W_iou_w, W_iou_b, U_iou_w, U_iou_b,
           U_f_w, U_f_b, lin_w, lin_b):
    del x, h, c

    embeds = _sc_gather(emb, wordid.astype(jnp.int32))

    U_cat = jnp.concatenate([U_f_w, U_iou_w],
                            axis=1).astype(jnp.bfloat16)
    b_cat = jnp.concatenate([U_f_b, U_iou_b]).reshape(1, -1)

    return _tree(embeds,
                 W_iou_w.astype(jnp.bfloat16), W_iou_b.reshape(1, -1),
                 U_cat, b_cat,
                 lin_w.astype(jnp.bfloat16), lin_b.reshape(1, -1))


# speedup vs baseline: 7.8492x; 1.0057x over previous
import functools

import jax
import jax.numpy as jnp
from jax import lax
from jax.experimental import pallas as pl
from jax.experimental.pallas import tpu as pltpu
from jax.experimental.pallas import tpu_sc as plsc

H = 256
N_LEAVES = 16384
DEPTH = 8
LEVEL_SIZES = [N_LEAVES >> l for l in range(DEPTH)]
N_TOTAL = sum(LEVEL_SIZES)
LEAF_BLK = 512
LVL_BLK = 512


def _sc_gather(table, idx):
    B = idx.shape[0]
    D = table.shape[1]
    NW = 32
    b_per_w = B // NW
    CH = 128
    n_ch = b_per_w // CH

    mesh = plsc.VectorSubcoreMesh(core_axis_name="c", subcore_axis_name="s")

    @functools.partial(
        pl.kernel, mesh=mesh,
        out_type=jax.ShapeDtypeStruct((B, D), jnp.float32),
        scratch_types=[
            pltpu.VMEM((CH,), jnp.int32),
            pltpu.VMEM((CH, D), jnp.float32),
            pltpu.SemaphoreType.DMA,
        ],
    )
    def k(table_hbm, idx_hbm, out_hbm, idx_v, rows_v, sem):
        wid = lax.axis_index("s") * 2 + lax.axis_index("c")
        for ch in range(n_ch):
            base = wid * b_per_w + ch * CH
            pltpu.sync_copy(idx_hbm.at[pl.ds(base, CH)], idx_v)
            pltpu.async_copy(table_hbm.at[idx_v], rows_v, sem).wait()
            pltpu.sync_copy(rows_v, out_hbm.at[pl.ds(base, CH)])

    return k(table, idx)


def _sig(x):
    return 0.5 * jnp.tanh(0.5 * x) + 0.5


def _tree_body(emb_hbm, wiou_ref, biou_ref, ucat_ref, bcat_ref, lw_ref,
               lb_ref, lg_ref, ebuf, hA, hB, cA, cB, sems):
    def emb_copy(i, slot):
        return pltpu.make_async_copy(
            emb_hbm.at[pl.ds(pl.multiple_of(i * LEAF_BLK, LEAF_BLK),
                             LEAF_BLK), :],
            ebuf.at[slot], sems.at[slot])

    def leaf_block(i, slot):
        emb_copy(i, slot).wait()
        e = ebuf[slot].astype(jnp.bfloat16)
        z = jnp.dot(e, wiou_ref[...],
                    preferred_element_type=jnp.float32) + biou_ref[...]
        ig = _sig(z[:, :H])
        og = _sig(z[:, H:2 * H])
        ug = jnp.tanh(z[:, 2 * H:])
        c0 = ig * ug
        h0 = (og * jnp.tanh(c0)).astype(jnp.bfloat16)
        half = pl.multiple_of(i * (LEAF_BLK // 2), LEAF_BLK // 2)
        hA[pl.ds(half, LEAF_BLK // 2), :] = h0.reshape(LEAF_BLK // 2, 2 * H)
        cA[pl.ds(half, LEAF_BLK // 2), :] = c0.reshape(LEAF_BLK // 2, 2 * H)
        lg_ref[pl.ds(pl.multiple_of(i * LEAF_BLK, LEAF_BLK),
                     LEAF_BLK), :] = jnp.dot(
            h0, lw_ref[...], preferred_element_type=jnp.float32) + lb_ref[...]

    n_leaf_blk = N_LEAVES // LEAF_BLK
    emb_copy(0, 0).start()

    def leaf_pair(p, carry):
        i0 = 2 * p
        emb_copy(i0 + 1, 1).start()
        leaf_block(i0, 0)

        @pl.when(i0 + 2 < n_leaf_blk)
        def _():
            emb_copy(i0 + 2, 0).start()
        leaf_block(i0 + 1, 1)
        return carry

    lax.fori_loop(0, n_leaf_blk // 2, leaf_pair, 0)

    def level_block(h_in, c_in, h_out, c_out, blk, jb, off, last):
        jb = pl.multiple_of(jb, blk)
        h2 = h_in[pl.ds(jb, blk), :]
        c2 = c_in[pl.ds(jb, blk), :]
        z = jnp.dot(h2, ucat_ref[...],
                    preferred_element_type=jnp.float32) + bcat_ref[...]
        f = _sig(z[:, :2 * H])
        cf = f[:, :H] * c2[:, :H] + f[:, H:] * c2[:, H:]
        ig = _sig(z[:, 2 * H:3 * H])
        og = _sig(z[:, 3 * H:4 * H])
        ug = jnp.tanh(z[:, 4 * H:])
        c_new = ig * ug + cf
        h_new = (og * jnp.tanh(c_new)).astype(jnp.bfloat16)
        if not last:
            jh = pl.multiple_of(jb // 2, blk // 2)
            h_out[pl.ds(jh, blk // 2), :] = h_new.reshape(blk // 2, 2 * H)
            c_out[pl.ds(jh, blk // 2), :] = c_new.reshape(blk // 2, 2 * H)
        lg_ref[pl.ds(pl.multiple_of(off + jb, blk), blk), :] = jnp.dot(
            h_new, lw_ref[...],
            preferred_element_type=jnp.float32) + lb_ref[...]

    off = N_LEAVES
    bufs = [(hA, cA), (hB, cB)]
    for l in range(1, DEPTH):
        n_l = LEVEL_SIZES[l]
        h_in, c_in = bufs[(l - 1) % 2]
        h_out, c_out = bufs[l % 2]
        blk = min(n_l, LVL_BLK)
        n_blk = n_l // blk
        last = l == DEPTH - 1
        off_l = off
        if n_blk > 1:
            def level_step(j, carry, h_in=h_in, c_in=c_in, h_out=h_out,
                           c_out=c_out, blk=blk, off_l=off_l, last=last):
                level_block(h_in, c_in, h_out, c_out, blk, j * blk,
                            off_l, last)
                return carry
            lax.fori_loop(0, n_blk, level_step, 0)
        else:
            level_block(h_in, c_in, h_out, c_out, blk, 0, off_l, last)
        off += n_l


def _tree(embeds, W_iou_w, W_iou_b, U_cat, b_cat, lin_w, lin_b):
    NC = lin_w.shape[1]
    vmem = pl.BlockSpec(memory_space=pltpu.MemorySpace.VMEM)
    return pl.pallas_call(
        _tree_body,
        in_specs=[
            pl.BlockSpec(memory_space=pltpu.MemorySpace.HBM),
            vmem, vmem, vmem, vmem, vmem, vmem,
        ],
        out_specs=pl.BlockSpec(memory_space=pltpu.MemorySpace.VMEM),
        out_shape=jax.ShapeDtypeStruct((N_TOTAL, NC), jnp.float32),
        scratch_shapes=[
            pltpu.VMEM((2, LEAF_BLK, H), jnp.float32),
            pltpu.VMEM((N_LEAVES // 2, 2 * H), jnp.bfloat16),
            pltpu.VMEM((N_LEAVES // 4, 2 * H), jnp.bfloat16),
            pltpu.VMEM((N_LEAVES // 2, 2 * H), jnp.float32),
            pltpu.VMEM((N_LEAVES // 4, 2 * H), jnp.float32),
            pltpu.SemaphoreType.DMA((2,)),
        ],
    )(embeds, W_iou_w, W_iou_b, U_cat, b_cat, lin_w, lin_b)


def kernel(wordid, x, h, c, emb, W_iou_w, W_iou_b, U_iou_w, U_iou_b,
           U_f_w, U_f_b, lin_w, lin_b):
    del x, h, c

    embeds = _sc_gather(emb, wordid.astype(jnp.int32))

    U_cat = jnp.concatenate([U_f_w, U_iou_w],
                            axis=1).astype(jnp.bfloat16)
    b_cat = jnp.concatenate([U_f_b, U_iou_b]).reshape(1, -1)

    return _tree(embeds,
                 W_iou_w.astype(jnp.bfloat16), W_iou_b.reshape(1, -1),
                 U_cat, b_cat,
                 lin_w.astype(jnp.bfloat16), lin_b.reshape(1, -1))
